# baseline (speedup 1.0000x reference)
import functools

import jax
import jax.numpy as jnp
from jax import lax
from jax.experimental import pallas as pl
from jax.experimental.pallas import tpu as pltpu
from jax.experimental.pallas import tpu_sc as plsc

NVX = 128
NVY = 128
VIEWS = 180
NDETU = 512
EXTENT = 2
CHANNEL = 8
K = VIEWS * NDETU
NVOX = NVX * NVY
SEG = VIEWS * EXTENT
SCALE = 2.0 * 3.14159265358979323846 / (2.0 * VIEWS * EXTENT)

NTILES = 32
NPAIR = CHANNEL // 2
NRANGES = NTILES // NPAIR
VPR = NVOX // NRANGES
GVOX = 32
NGROUPS = VPR // GVOX
CHUNK = GVOX * SEG


def _bp_kernel(x_hbm, idx_hbm, out_hbm, table_v, idx_a, idx_b, outa_v,
               outb_v, sem_a, sem_b):
    c = lax.axis_index("c")
    s = lax.axis_index("s")
    wid = s * 2 + c
    pair = wid % NPAIR
    rng = wid // NPAIR
    tile_vox0 = rng * VPR
    idx_base = tile_vox0 * SEG

    pltpu.sync_copy(x_hbm.at[pair], table_v)

    bufs = (idx_a, idx_b)
    sems = (sem_a, sem_b)

    def start_fetch(g, buf, sem):
        pltpu.make_async_copy(
            idx_hbm.at[pl.ds(idx_base + g * CHUNK, CHUNK)], buf, sem
        ).start()

    def wait_fetch(g, buf, sem):
        pltpu.make_async_copy(
            idx_hbm.at[pl.ds(idx_base + g * CHUNK, CHUNK)], buf, sem
        ).wait()

    start_fetch(0, idx_a, sem_a)

    lane = jax.lax.iota(jnp.int32, 16)

    def compute_group(g, buf):
        def vg_body(vg, _):
            pos0 = (vg * 16 + lane) * SEG + lane

            def gather_step(pos, accs):
                acca, accb = accs
                packed = plsc.load_gather(table_v,
                                          [plsc.load_gather(buf, [pos])])
                va = plsc.bitcast(
                    lax.shift_left(packed, jnp.int32(16)), jnp.float32)
                vb = plsc.bitcast(packed & jnp.int32(-65536), jnp.float32)
                return (acca + va, accb + vb)

            def j_body(j, accs):
                j8 = j * 8
                for u in range(8):
                    accs = gather_step(pos0 + (j8 + u), accs)
                return accs

            z = jnp.zeros(16, jnp.float32)
            accs = lax.fori_loop(0, 344 // 8, j_body, (z, z))
            for j in range(344, SEG):
                wrap = jnp.where(lane >= SEG - j, SEG, 0)
                accs = gather_step(pos0 + j - wrap, accs)
            acca, accb = accs
            off = g * GVOX + vg * 16
            outa_v[pl.ds(off, 16)] = acca * SCALE
            outb_v[pl.ds(off, 16)] = accb * SCALE
            return 0

        lax.fori_loop(0, GVOX // 16, vg_body, 0)

    def group_pair_body(k, _):
        for b in range(2):
            g = k * 2 + b
            wait_fetch(g, bufs[b], sems[b])

            @pl.when(g + 1 < NGROUPS)
            def _():
                start_fetch(g + 1, bufs[1 - b], sems[1 - b])

            compute_group(g, bufs[b])
        return 0

    lax.fori_loop(0, NGROUPS // 2, group_pair_body, 0)

    pltpu.sync_copy(outa_v, out_hbm.at[pair * 2, pl.ds(tile_vox0, VPR)])
    pltpu.sync_copy(outb_v, out_hbm.at[pair * 2 + 1, pl.ds(tile_vox0, VPR)])


@jax.jit
def _backproj(xp, indices):
    f = functools.partial(
        pl.kernel,
        mesh=plsc.VectorSubcoreMesh(core_axis_name="c", subcore_axis_name="s"),
        out_type=jax.ShapeDtypeStruct((CHANNEL, NVOX), jnp.float32),
        compiler_params=pltpu.CompilerParams(needs_layout_passes=False),
        scratch_types=[
            pltpu.VMEM((K,), jnp.int32),
            pltpu.VMEM((CHUNK,), jnp.int32),
            pltpu.VMEM((CHUNK,), jnp.int32),
            pltpu.VMEM((VPR,), jnp.float32),
            pltpu.VMEM((VPR,), jnp.float32),
            pltpu.SemaphoreType.DMA,
            pltpu.SemaphoreType.DMA,
        ],
    )(_bp_kernel)
    return f(xp, indices)


def kernel(input, indices):
    u = input.reshape(CHANNEL, K).view(jnp.uint32)
    half = jnp.uint32(0x8000)
    lo = (u[0::2] + half) >> 16
    hi = (u[1::2] + half) & jnp.uint32(0xFFFF0000)
    packed = (lo | hi).view(jnp.int32)
    out = _backproj(packed, indices)
    return out.reshape(1, CHANNEL, NVX, NVY)


# speedup vs baseline: 1.1091x; 1.1091x over previous
import functools

import jax
import jax.numpy as jnp
from jax import lax
from jax.experimental import pallas as pl
from jax.experimental.pallas import tpu as pltpu
from jax.experimental.pallas import tpu_sc as plsc

NVX = 128
NVY = 128
VIEWS = 180
NDETU = 512
EXTENT = 2
CHANNEL = 8
K = VIEWS * NDETU
NVOX = NVX * NVY
SEG = VIEWS * EXTENT
SCALE = 2.0 * 3.14159265358979323846 / (2.0 * VIEWS * EXTENT)

NTILES = 32
NPAIR = CHANNEL // 2
NRANGES = NTILES // NPAIR
VPR = NVOX // NRANGES
GVOX = 32
NGROUPS = VPR // GVOX
CHUNK = GVOX * SEG


def _bp_kernel(x_hbm, idx_hbm, out_hbm, table_v, idx_a, idx_b, outa_v,
               outb_v, sem_a, sem_b, sem_t):
    c = lax.axis_index("c")
    s = lax.axis_index("s")
    wid = s * 2 + c
    pair = wid % NPAIR
    rng = wid // NPAIR
    tile_vox0 = rng * VPR
    idx_base = tile_vox0 * SEG

    Q = K // 4
    for q in range(4):
        pltpu.make_async_copy(x_hbm.at[pair, pl.ds(q * Q, Q)],
                              table_v.at[pl.ds(q * Q, Q)], sem_t).start()
    for q in range(4):
        pltpu.make_async_copy(x_hbm.at[pair, pl.ds(q * Q, Q)],
                              table_v.at[pl.ds(q * Q, Q)], sem_t).wait()

    bufs = (idx_a, idx_b)
    sems = (sem_a, sem_b)

    H = CHUNK // 2

    def start_fetch(g, buf, sem):
        for h in range(2):
            pltpu.make_async_copy(
                idx_hbm.at[pl.ds(idx_base + g * CHUNK + h * H, H)],
                buf.at[pl.ds(h * H, H)], sem).start()

    def wait_fetch(g, buf, sem):
        for h in range(2):
            pltpu.make_async_copy(
                idx_hbm.at[pl.ds(idx_base + g * CHUNK + h * H, H)],
                buf.at[pl.ds(h * H, H)], sem).wait()

    start_fetch(0, idx_a, sem_a)

    lane = jax.lax.iota(jnp.int32, 16)

    def compute_group(g, buf):
        def vg_body(vg, _):
            pos0 = (vg * 16 + lane) * SEG + lane

            def gather_step(pos, accs):
                acca, accb = accs
                packed = plsc.load_gather(table_v,
                                          [plsc.load_gather(buf, [pos])])
                va = plsc.bitcast(
                    lax.shift_left(packed, jnp.int32(16)), jnp.float32)
                vb = plsc.bitcast(packed & jnp.int32(-65536), jnp.float32)
                return (acca + va, accb + vb)

            def j_body(j, accs):
                j8 = j * 8
                for u in range(8):
                    accs = gather_step(pos0 + (j8 + u), accs)
                return accs

            z = jnp.zeros(16, jnp.float32)
            accs = lax.fori_loop(0, 344 // 8, j_body, (z, z))
            for j in range(344, SEG):
                wrap = jnp.where(lane >= SEG - j, SEG, 0)
                accs = gather_step(pos0 + j - wrap, accs)
            acca, accb = accs
            off = g * GVOX + vg * 16
            outa_v[pl.ds(off, 16)] = acca * SCALE
            outb_v[pl.ds(off, 16)] = accb * SCALE
            return 0

        lax.fori_loop(0, GVOX // 16, vg_body, 0)

    def group_pair_body(k, _):
        for b in range(2):
            g = k * 2 + b
            wait_fetch(g, bufs[b], sems[b])

            @pl.when(g + 1 < NGROUPS)
            def _():
                start_fetch(g + 1, bufs[1 - b], sems[1 - b])

            compute_group(g, bufs[b])
        return 0

    lax.fori_loop(0, NGROUPS // 2, group_pair_body, 0)

    pltpu.sync_copy(outa_v, out_hbm.at[pair * 2, pl.ds(tile_vox0, VPR)])
    pltpu.sync_copy(outb_v, out_hbm.at[pair * 2 + 1, pl.ds(tile_vox0, VPR)])


@jax.jit
def _backproj(xp, indices):
    f = functools.partial(
        pl.kernel,
        mesh=plsc.VectorSubcoreMesh(core_axis_name="c", subcore_axis_name="s"),
        out_type=jax.ShapeDtypeStruct((CHANNEL, NVOX), jnp.float32),
        compiler_params=pltpu.CompilerParams(needs_layout_passes=False),
        scratch_types=[
            pltpu.VMEM((K,), jnp.int32),
            pltpu.VMEM((CHUNK,), jnp.int32),
            pltpu.VMEM((CHUNK,), jnp.int32),
            pltpu.VMEM((VPR,), jnp.float32),
            pltpu.VMEM((VPR,), jnp.float32),
            pltpu.SemaphoreType.DMA,
            pltpu.SemaphoreType.DMA,
            pltpu.SemaphoreType.DMA,
        ],
    )(_bp_kernel)
    return f(xp, indices)


def kernel(input, indices):
    x = input.reshape(CHANNEL, K)
    h = x.astype(jnp.bfloat16).view(jnp.uint16).astype(jnp.uint32)
    hh = h.reshape(NPAIR, 2, K)
    packed = (hh[:, 0] | (hh[:, 1] << 16)).view(jnp.int32)
    out = _backproj(packed, indices)
    return out.reshape(1, CHANNEL, NVX, NVY)


# speedup vs baseline: 1.1942x; 1.0768x over previous
import functools

import jax
import jax.numpy as jnp
from jax import lax
from jax.experimental import pallas as pl
from jax.experimental.pallas import tpu as pltpu
from jax.experimental.pallas import tpu_sc as plsc

NVX = 128
NVY = 128
VIEWS = 180
NDETU = 512
EXTENT = 2
CHANNEL = 8
K = VIEWS * NDETU
NVOX = NVX * NVY
SEG = VIEWS * EXTENT
SCALE = 2.0 * 3.14159265358979323846 / (2.0 * VIEWS * EXTENT)

NTILES = 32
NPAIR = CHANNEL // 2
NRANGES = NTILES // NPAIR
VPR = NVOX // NRANGES
GVOX = 32
NGROUPS = VPR // GVOX
CHUNK = GVOX * SEG


def _bp_kernel(x_hbm, idx_hbm, out_hbm, table_v, idx_a, idx_b, outa_v,
               outb_v, sem_a, sem_b):
    c = lax.axis_index("c")
    s = lax.axis_index("s")
    wid = s * 2 + c
    pair = wid % NPAIR
    rng = wid // NPAIR
    tile_vox0 = rng * VPR
    idx_base = tile_vox0 * SEG

    pltpu.sync_copy(x_hbm.at[pair], table_v)

    bufs = (idx_a, idx_b)
    sems = (sem_a, sem_b)

    def start_fetch(g, buf, sem):
        pltpu.make_async_copy(
            idx_hbm.at[pl.ds(idx_base + g * CHUNK, CHUNK)], buf, sem
        ).start()

    def wait_fetch(g, buf, sem):
        pltpu.make_async_copy(
            idx_hbm.at[pl.ds(idx_base + g * CHUNK, CHUNK)], buf, sem
        ).wait()

    start_fetch(0, idx_a, sem_a)

    lane = jax.lax.iota(jnp.int32, 16)

    def compute_group(g, buf):
        def vg_body(vg, _):
            pos0 = (vg * 16 + lane) * SEG + lane

            def gather_step(pos, accs):
                acca, accb = accs
                packed = plsc.load_gather(table_v,
                                          [plsc.load_gather(buf, [pos])])
                va = plsc.bitcast(
                    lax.shift_left(packed, jnp.int32(16)), jnp.float32)
                vb = plsc.bitcast(packed & jnp.int32(-65536), jnp.float32)
                return (acca + va, accb + vb)

            def j_body(j, accs):
                j8 = j * 8
                for u in range(8):
                    accs = gather_step(pos0 + (j8 + u), accs)
                return accs

            z = jnp.zeros(16, jnp.float32)
            accs = lax.fori_loop(0, 344 // 8, j_body, (z, z))
            for j in range(344, SEG):
                wrap = jnp.where(lane >= SEG - j, SEG, 0)
                accs = gather_step(pos0 + j - wrap, accs)
            acca, accb = accs
            off = g * GVOX + vg * 16
            outa_v[pl.ds(off, 16)] = acca * SCALE
            outb_v[pl.ds(off, 16)] = accb * SCALE
            return 0

        lax.fori_loop(0, GVOX // 16, vg_body, 0)

    def group_pair_body(k, _):
        for b in range(2):
            g = k * 2 + b
            wait_fetch(g, bufs[b], sems[b])

            @pl.when(g + 1 < NGROUPS)
            def _():
                start_fetch(g + 1, bufs[1 - b], sems[1 - b])

            compute_group(g, bufs[b])
        return 0

    lax.fori_loop(0, NGROUPS // 2, group_pair_body, 0)

    pltpu.sync_copy(outa_v, out_hbm.at[pair, pl.ds(tile_vox0, VPR)])
    pltpu.sync_copy(outb_v, out_hbm.at[pair + 4, pl.ds(tile_vox0, VPR)])


@jax.jit
def _backproj(xp, indices):
    f = functools.partial(
        pl.kernel,
        mesh=plsc.VectorSubcoreMesh(core_axis_name="c", subcore_axis_name="s"),
        out_type=jax.ShapeDtypeStruct((CHANNEL, NVOX), jnp.float32),
        compiler_params=pltpu.CompilerParams(needs_layout_passes=False),
        scratch_types=[
            pltpu.VMEM((K,), jnp.int32),
            pltpu.VMEM((CHUNK,), jnp.int32),
            pltpu.VMEM((CHUNK,), jnp.int32),
            pltpu.VMEM((VPR,), jnp.float32),
            pltpu.VMEM((VPR,), jnp.float32),
            pltpu.SemaphoreType.DMA,
            pltpu.SemaphoreType.DMA,
        ],
    )(_bp_kernel)
    return f(xp, indices)


def kernel(input, indices):
    x = input.reshape(CHANNEL, K)
    h = x.astype(jnp.bfloat16).view(jnp.uint16).astype(jnp.uint32)
    packed = (h[:NPAIR] | (h[NPAIR:] << 16)).view(jnp.int32)
    out = _backproj(packed, indices)
    return out.reshape(1, CHANNEL, NVX, NVY)


# speedup vs baseline: 1.2362x; 1.0351x over previous
import functools

import jax
import jax.numpy as jnp
from jax import lax
from jax.experimental import pallas as pl
from jax.experimental.pallas import tpu as pltpu
from jax.experimental.pallas import tpu_sc as plsc

NVX = 128
NVY = 128
VIEWS = 180
NDETU = 512
EXTENT = 2
CHANNEL = 8
K = VIEWS * NDETU
NVOX = NVX * NVY
SEG = VIEWS * EXTENT
SCALE = 2.0 * 3.14159265358979323846 / (2.0 * VIEWS * EXTENT)

NTILES = 32
NPAIR = CHANNEL // 2
NRANGES = NTILES // NPAIR
VPR = NVOX // NRANGES
GVOX = 32
NGROUPS = VPR // GVOX
CHUNK = GVOX * SEG


def _bp_kernel(x_hbm, idx_hbm, out_hbm, table_v, idx_a, idx_b, idx_c,
               outa_v, outb_v, sem_a, sem_b, sem_c):
    c = lax.axis_index("c")
    s = lax.axis_index("s")
    wid = s * 2 + c
    pair = wid % NPAIR
    rng = wid // NPAIR
    tile_vox0 = rng * VPR
    idx_base = tile_vox0 * SEG

    bufs = (idx_a, idx_b, idx_c)
    sems = (sem_a, sem_b, sem_c)

    def start_fetch(g, buf, sem):
        pltpu.make_async_copy(
            idx_hbm.at[pl.ds(idx_base + g * CHUNK, CHUNK)], buf, sem
        ).start()

    def wait_fetch(g, buf, sem):
        pltpu.make_async_copy(
            idx_hbm.at[pl.ds(idx_base + g * CHUNK, CHUNK)], buf, sem
        ).wait()

    for g0 in range(3):
        start_fetch(g0, bufs[g0], sems[g0])
    pltpu.sync_copy(x_hbm.at[pair], table_v)

    lane = jax.lax.iota(jnp.int32, 16)

    def compute_group(g, buf):
        def vg_body(vg, _):
            pos0 = (vg * 16 + lane) * SEG + lane

            def gather_step(pos, accs):
                acca, accb = accs
                packed = plsc.load_gather(table_v,
                                          [plsc.load_gather(buf, [pos])])
                va = plsc.bitcast(
                    lax.shift_left(packed, jnp.int32(16)), jnp.float32)
                vb = plsc.bitcast(packed & jnp.int32(-65536), jnp.float32)
                return (acca + va, accb + vb)

            def j_body(j, accs):
                j8 = j * 8
                for u in range(8):
                    accs = gather_step(pos0 + (j8 + u), accs)
                return accs

            z = jnp.zeros(16, jnp.float32)
            accs = lax.fori_loop(0, 344 // 8, j_body, (z, z))
            for j in range(344, SEG):
                wrap = jnp.where(lane >= SEG - j, SEG, 0)
                accs = gather_step(pos0 + j - wrap, accs)
            acca, accb = accs
            off = g * GVOX + vg * 16
            outa_v[pl.ds(off, 16)] = acca * SCALE
            outb_v[pl.ds(off, 16)] = accb * SCALE
            return 0

        lax.fori_loop(0, GVOX // 16, vg_body, 0)

    def ring_body(k, _):
        for b in range(3):
            g = k * 3 + b
            wait_fetch(g, bufs[b], sems[b])
            compute_group(g, bufs[b])

            @pl.when(g + 3 < NGROUPS)
            def _():
                start_fetch(g + 3, bufs[b], sems[b])

        return 0

    lax.fori_loop(0, NGROUPS // 3, ring_body, 0)
    wait_fetch(NGROUPS - 1, bufs[0], sems[0])
    compute_group(NGROUPS - 1, bufs[0])

    pltpu.sync_copy(outa_v, out_hbm.at[pair, pl.ds(tile_vox0, VPR)])
    pltpu.sync_copy(outb_v, out_hbm.at[pair + 4, pl.ds(tile_vox0, VPR)])


@jax.jit
def _backproj(xp, indices):
    f = functools.partial(
        pl.kernel,
        mesh=plsc.VectorSubcoreMesh(core_axis_name="c", subcore_axis_name="s"),
        out_type=jax.ShapeDtypeStruct((CHANNEL, NVOX), jnp.float32),
        compiler_params=pltpu.CompilerParams(needs_layout_passes=False),
        scratch_types=[
            pltpu.VMEM((K,), jnp.int32),
            pltpu.VMEM((CHUNK,), jnp.int32),
            pltpu.VMEM((CHUNK,), jnp.int32),
            pltpu.VMEM((CHUNK,), jnp.int32),
            pltpu.VMEM((VPR,), jnp.float32),
            pltpu.VMEM((VPR,), jnp.float32),
            pltpu.SemaphoreType.DMA,
            pltpu.SemaphoreType.DMA,
            pltpu.SemaphoreType.DMA,
        ],
    )(_bp_kernel)
    return f(xp, indices)


def kernel(input, indices):
    x = input.reshape(CHANNEL, K)
    h = x.astype(jnp.bfloat16).view(jnp.uint16).astype(jnp.uint32)
    packed = (h[:NPAIR] | (h[NPAIR:] << 16)).view(jnp.int32)
    out = _backproj(packed, indices)
    return out.reshape(1, CHANNEL, NVX, NVY)


# speedup vs baseline: 1.2371x; 1.0008x over previous
import functools

import jax
import jax.numpy as jnp
from jax import lax
from jax.experimental import pallas as pl
from jax.experimental.pallas import tpu as pltpu
from jax.experimental.pallas import tpu_sc as plsc

NVX = 128
NVY = 128
VIEWS = 180
NDETU = 512
EXTENT = 2
CHANNEL = 8
K = VIEWS * NDETU
NVOX = NVX * NVY
SEG = VIEWS * EXTENT
SCALE = 2.0 * 3.14159265358979323846 / (2.0 * VIEWS * EXTENT)

NTILES = 32
NPAIR = CHANNEL // 2
NRANGES = NTILES // NPAIR
VPR = NVOX // NRANGES
GVOX = 32
NGROUPS = VPR // GVOX
CHUNK = GVOX * SEG


def _bp_kernel(x_hbm, idx_hbm, out_hbm, table_v, idx_a, idx_b, idx_c,
               outa_v, outb_v, sem_a, sem_b, sem_c):
    c = lax.axis_index("c")
    s = lax.axis_index("s")
    wid = s * 2 + c
    pair = wid % NPAIR
    rng = wid // NPAIR
    tile_vox0 = rng * VPR
    idx_base = tile_vox0 * SEG

    bufs = (idx_a, idx_b, idx_c)
    sems = (sem_a, sem_b, sem_c)

    def start_fetch(g, buf, sem):
        pltpu.make_async_copy(
            idx_hbm.at[pl.ds(idx_base + g * CHUNK, CHUNK)], buf, sem
        ).start()

    def wait_fetch(g, buf, sem):
        pltpu.make_async_copy(
            idx_hbm.at[pl.ds(idx_base + g * CHUNK, CHUNK)], buf, sem
        ).wait()

    for g0 in range(3):
        start_fetch(g0, bufs[g0], sems[g0])
    pltpu.sync_copy(x_hbm.at[pair], table_v)

    lane = jax.lax.iota(jnp.int32, 16)

    def compute_group(g, buf):
        def vg_body(vg, _):
            pos0 = (vg * 16 + lane) * SEG + lane

            def gather_step(pos, accs, k):
                accs = list(accs)
                packed = plsc.load_gather(table_v,
                                          [plsc.load_gather(buf, [pos])])
                va = plsc.bitcast(
                    lax.shift_left(packed, jnp.int32(16)), jnp.float32)
                vb = plsc.bitcast(packed & jnp.int32(-65536), jnp.float32)
                accs[k] = accs[k] + va
                accs[2 + k] = accs[2 + k] + vb
                return tuple(accs)

            def j_body(j, accs):
                j15 = j * 15
                for u in range(15):
                    accs = gather_step(pos0 + (j15 + u), accs, u & 1)
                return accs

            z = jnp.zeros(16, jnp.float32)
            accs = lax.fori_loop(0, 345 // 15, j_body, (z, z, z, z))
            for j in range(345, SEG):
                wrap = jnp.where(lane >= SEG - j, SEG, 0)
                accs = gather_step(pos0 + j - wrap, accs, j & 1)
            acca, accb = accs[0] + accs[1], accs[2] + accs[3]
            off = g * GVOX + vg * 16
            outa_v[pl.ds(off, 16)] = acca * SCALE
            outb_v[pl.ds(off, 16)] = accb * SCALE
            return 0

        lax.fori_loop(0, GVOX // 16, vg_body, 0)

    def ring_body(k, _):
        for b in range(3):
            g = k * 3 + b
            wait_fetch(g, bufs[b], sems[b])
            compute_group(g, bufs[b])

            @pl.when(g + 3 < NGROUPS)
            def _():
                start_fetch(g + 3, bufs[b], sems[b])

        return 0

    lax.fori_loop(0, NGROUPS // 3, ring_body, 0)
    wait_fetch(NGROUPS - 1, bufs[0], sems[0])
    compute_group(NGROUPS - 1, bufs[0])

    pltpu.sync_copy(outa_v, out_hbm.at[pair, pl.ds(tile_vox0, VPR)])
    pltpu.sync_copy(outb_v, out_hbm.at[pair + 4, pl.ds(tile_vox0, VPR)])


@jax.jit
def _backproj(xp, indices):
    f = functools.partial(
        pl.kernel,
        mesh=plsc.VectorSubcoreMesh(core_axis_name="c", subcore_axis_name="s"),
        out_type=jax.ShapeDtypeStruct((CHANNEL, NVOX), jnp.float32),
        compiler_params=pltpu.CompilerParams(needs_layout_passes=False),
        scratch_types=[
            pltpu.VMEM((K,), jnp.int32),
            pltpu.VMEM((CHUNK,), jnp.int32),
            pltpu.VMEM((CHUNK,), jnp.int32),
            pltpu.VMEM((CHUNK,), jnp.int32),
            pltpu.VMEM((VPR,), jnp.float32),
            pltpu.VMEM((VPR,), jnp.float32),
            pltpu.SemaphoreType.DMA,
            pltpu.SemaphoreType.DMA,
            pltpu.SemaphoreType.DMA,
        ],
    )(_bp_kernel)
    return f(xp, indices)


def kernel(input, indices):
    x = input.reshape(CHANNEL, K)
    h = x.astype(jnp.bfloat16).view(jnp.uint16).astype(jnp.uint32)
    packed = (h[:NPAIR] | (h[NPAIR:] << 16)).view(jnp.int32)
    out = _backproj(packed, indices)
    return out.reshape(1, CHANNEL, NVX, NVY)
